# Initial kernel scaffold; baseline (speedup 1.0000x reference)
import functools

import jax
import jax.numpy as jnp
from jax import lax
from jax.experimental import pallas as pl
from jax.experimental.pallas import tpu as pltpu
from jax.experimental.pallas import tpu_sc as plsc

MAXLEN = 200
EMBED = 64
SEQ_SPLIT = 2
SUB = MAXLEN // SEQ_SPLIT


def kernel(x, token_table, pos_table):
    B, T = x.shape
    V, D = token_table.shape
    assert T == MAXLEN and D == EMBED

    info = plsc.get_sparse_core_info()
    nw = info.num_cores * info.num_subcores
    rows_per_w = B // nw

    x3 = x.astype(jnp.int32).reshape(B, SEQ_SPLIT, SUB)
    pos_idx = jnp.arange(T, dtype=jnp.int32).reshape(SEQ_SPLIT, SUB)

    mesh = plsc.VectorSubcoreMesh(core_axis_name="c", subcore_axis_name="s")

    @functools.partial(
        pl.kernel,
        mesh=mesh,
        out_type=jax.ShapeDtypeStruct((B, SEQ_SPLIT, SUB, D), jnp.float32),
        scratch_types=[
            pltpu.VMEM((SEQ_SPLIT, SUB), jnp.int32),
            pltpu.VMEM((SEQ_SPLIT, SUB), jnp.int32),
            pltpu.VMEM((SEQ_SPLIT, SUB, D), jnp.float32),
            pltpu.SemaphoreType.DMA,
        ],
    )
    def sc_kernel(x_hbm, tok_hbm, pos_hbm, pidx_hbm, out_hbm,
                  idx_v, pidx_v, rows_v, sem):
        cid = lax.axis_index("c")
        sid = lax.axis_index("s")
        wid = sid * info.num_cores + cid
        base = wid * rows_per_w

        pltpu.sync_copy(pidx_hbm, pidx_v)

        def body(r, carry):
            row = base + r
            pltpu.sync_copy(x_hbm.at[row], idx_v)
            pltpu.async_copy(tok_hbm.at[idx_v], rows_v, sem).wait()
            pltpu.async_copy(pos_hbm.at[pidx_v], rows_v, sem, add=True).wait()
            pltpu.sync_copy(rows_v, out_hbm.at[row])
            return carry

        lax.fori_loop(0, rows_per_w, body, 0)

    out = sc_kernel(x3, token_table, pos_table, pos_idx)
    return out.reshape(B, T, D)


# speedup vs baseline: 2.2763x; 2.2763x over previous
import functools

import jax
import jax.numpy as jnp
from jax import lax
from jax.experimental import pallas as pl
from jax.experimental.pallas import tpu as pltpu
from jax.experimental.pallas import tpu_sc as plsc

MAXLEN = 200
EMBED = 64
SEQ_SPLIT = 2
SUB = MAXLEN // SEQ_SPLIT


def kernel(x, token_table, pos_table):
    B, T = x.shape
    V, D = token_table.shape
    assert T == MAXLEN and D == EMBED

    info = plsc.get_sparse_core_info()
    nw = info.num_cores * info.num_subcores
    rows_per_w = B // nw

    n_chunks = B * SEQ_SPLIT
    chunks_per_w = n_chunks // nw

    x2 = x.astype(jnp.int32).reshape(n_chunks, SUB)
    pos_idx = jnp.arange(T, dtype=jnp.int32).reshape(SEQ_SPLIT, SUB)

    mesh = plsc.VectorSubcoreMesh(core_axis_name="c", subcore_axis_name="s")

    @functools.partial(
        pl.kernel,
        mesh=mesh,
        out_type=jax.ShapeDtypeStruct((n_chunks, SUB, D), jnp.float32),
        scratch_types=[
            pltpu.VMEM((SUB,), jnp.int32),
            pltpu.VMEM((SEQ_SPLIT, SUB), jnp.int32),
            pltpu.VMEM((SUB, D), jnp.float32),
            pltpu.SemaphoreType.DMA,
        ],
        compiler_params=pltpu.CompilerParams(use_tc_tiling_on_sc=False),
    )
    def sc_kernel(x_hbm, tok_hbm, pos_hbm, pidx_hbm, out_hbm,
                  idx_v, pidx_v, rows_v, sem):
        cid = lax.axis_index("c")
        sid = lax.axis_index("s")
        wid = sid * info.num_cores + cid
        base = wid * chunks_per_w

        pltpu.sync_copy(pidx_hbm, pidx_v)

        def body(r, carry):
            chunk = base + r
            pltpu.sync_copy(x_hbm.at[chunk], idx_v)
            pltpu.async_copy(tok_hbm.at[idx_v], rows_v, sem).wait()
            pltpu.async_copy(
                pos_hbm.at[pidx_v.at[chunk % SEQ_SPLIT]], rows_v, sem, add=True
            ).wait()
            pltpu.sync_copy(rows_v, out_hbm.at[chunk])
            return carry

        lax.fori_loop(0, chunks_per_w, body, 0)

    out = sc_kernel(x2, token_table, pos_table, pos_idx)
    return out.reshape(B, T, D)


# speedup vs baseline: 3.9254x; 1.7245x over previous
import functools

import jax
import jax.numpy as jnp
from jax import lax
from jax.experimental import pallas as pl
from jax.experimental.pallas import tpu as pltpu
from jax.experimental.pallas import tpu_sc as plsc

MAXLEN = 200
EMBED = 64
SUB = 100
K = 4
ROWS = K * SUB


def kernel(x, token_table, pos_table):
    B, T = x.shape
    V, D = token_table.shape
    assert T == MAXLEN and D == EMBED

    info = plsc.get_sparse_core_info()
    nw = info.num_cores * info.num_subcores
    n_supers = (B * T) // ROWS
    supers_per_w = n_supers // nw

    x3 = x.astype(jnp.int32).reshape(n_supers, K, SUB)

    mesh = plsc.VectorSubcoreMesh(core_axis_name="c", subcore_axis_name="s")

    @functools.partial(
        pl.kernel,
        mesh=mesh,
        out_type=jax.ShapeDtypeStruct((B * T, D), jnp.float32),
        scratch_types=[
            pltpu.VMEM_SHARED((ROWS, D), jnp.float32),
            pltpu.VMEM((ROWS, D), jnp.float32),
            pltpu.VMEM((ROWS, D), jnp.float32),
            pltpu.VMEM((K, SUB), jnp.int32),
            pltpu.VMEM((K, SUB), jnp.int32),
            pltpu.SemaphoreType.DMA,
            pltpu.SemaphoreType.DMA,
            pltpu.SemaphoreType.DMA,
            pltpu.SemaphoreType.DMA,
        ],
        compiler_params=pltpu.CompilerParams(use_tc_tiling_on_sc=False),
    )
    def sc_kernel(x_hbm, tok_hbm, pos_hbm, out_hbm,
                  posfill, rows0, rows1, idx0, idx1,
                  sem_g0, sem_g1, sem_s0, sem_s1):
        rows = (rows0, rows1)
        idx = (idx0, idx1)
        sem_g = (sem_g0, sem_g1)
        sem_s = (sem_s0, sem_s1)

        cid = lax.axis_index("c")
        sid = lax.axis_index("s")
        wid = sid * info.num_cores + cid
        base = wid * supers_per_w

        @pl.when(sid == 0)
        def _():
            pltpu.sync_copy(pos_hbm, rows0.at[pl.ds(0, MAXLEN)])
            for rep in range(ROWS // MAXLEN):
                pltpu.sync_copy(rows0.at[pl.ds(0, MAXLEN)],
                                posfill.at[pl.ds(rep * MAXLEN, MAXLEN)])
        plsc.subcore_barrier()

        def wait_store(b):
            pltpu.make_async_copy(
                rows[b], out_hbm.at[pl.ds(0, ROWS)], sem_s[b]).wait()

        def drain_gathers(b):
            for j in range(K):
                pltpu.make_async_copy(
                    tok_hbm.at[idx[b].at[j]],
                    rows[b].at[pl.ds(j * SUB, SUB)],
                    sem_g[b]).wait()

        def stage_a(i, b, first_use):
            if not first_use:
                wait_store(b)
            pltpu.sync_copy(posfill, rows[b])
            pltpu.sync_copy(x_hbm.at[base + i], idx[b])
            for j in range(K):
                pltpu.async_copy(
                    tok_hbm.at[idx[b].at[j]],
                    rows[b].at[pl.ds(j * SUB, SUB)],
                    sem_g[b], add=True)

        def stage_b(i, b):
            drain_gathers(b)
            pltpu.async_copy(
                rows[b], out_hbm.at[pl.ds((base + i) * ROWS, ROWS)], sem_s[b])

        stage_a(0, 0, True)
        stage_a(1, 1, True)
        stage_b(0, 0)

        @pl.loop(0, (supers_per_w - 2) // 2)
        def _(t):
            i = 2 + 2 * t
            stage_a(i, 0, False)
            stage_b(i - 1, 1)
            stage_a(i + 1, 1, False)
            stage_b(i, 0)

        stage_b(supers_per_w - 1, 1)
        wait_store(0)
        wait_store(1)

    out = sc_kernel(x3, token_table, pos_table)
    return out.reshape(B, T, D)


# speedup vs baseline: 3.9608x; 1.0090x over previous
import functools

import jax
import jax.numpy as jnp
from jax import lax
from jax.experimental import pallas as pl
from jax.experimental.pallas import tpu as pltpu
from jax.experimental.pallas import tpu_sc as plsc

MAXLEN = 200
EMBED = 64
SEQ_PER_SUPER = 2


def kernel(x, token_table, pos_table):
    B, T = x.shape
    V, D = token_table.shape
    assert T == MAXLEN and D == EMBED

    info = plsc.get_sparse_core_info()
    nw = info.num_cores * info.num_subcores
    supers_per_w = B // (SEQ_PER_SUPER * nw)

    x32 = x.astype(jnp.int32)

    mesh = plsc.VectorSubcoreMesh(core_axis_name="c", subcore_axis_name="s")

    @functools.partial(
        pl.kernel,
        mesh=mesh,
        out_type=jax.ShapeDtypeStruct((B, T, D), jnp.float32),
        scratch_types=[
            pltpu.VMEM_SHARED((SEQ_PER_SUPER, MAXLEN, D), jnp.float32),
            pltpu.VMEM((SEQ_PER_SUPER, MAXLEN, D), jnp.float32),
            pltpu.VMEM((SEQ_PER_SUPER, MAXLEN, D), jnp.float32),
            pltpu.VMEM((SEQ_PER_SUPER, MAXLEN), jnp.int32),
            pltpu.VMEM((SEQ_PER_SUPER, MAXLEN), jnp.int32),
            pltpu.SemaphoreType.DMA,
            pltpu.SemaphoreType.DMA,
            pltpu.SemaphoreType.DMA,
            pltpu.SemaphoreType.DMA,
        ],
        compiler_params=pltpu.CompilerParams(use_tc_tiling_on_sc=False),
    )
    def sc_kernel(x_hbm, tok_hbm, pos_hbm, out_hbm,
                  posfill, rows0, rows1, idx0, idx1,
                  sem_g0, sem_g1, sem_s0, sem_s1):
        rows = (rows0, rows1)
        idx = (idx0, idx1)
        sem_g = (sem_g0, sem_g1)
        sem_s = (sem_s0, sem_s1)

        cid = lax.axis_index("c")
        sid = lax.axis_index("s")
        wid = sid * info.num_cores + cid
        base = wid * supers_per_w

        @pl.when(sid == 0)
        def _():
            pltpu.sync_copy(pos_hbm, rows0.at[0])
            for rep in range(SEQ_PER_SUPER):
                pltpu.sync_copy(rows0.at[0], posfill.at[rep])
        plsc.subcore_barrier()

        def wait_store(b):
            pltpu.make_async_copy(
                rows[b], out_hbm.at[pl.ds(0, SEQ_PER_SUPER)], sem_s[b]).wait()

        def stage_a(i, b, first_use):
            if not first_use:
                wait_store(b)
            pltpu.sync_copy(posfill, rows[b])
            pltpu.sync_copy(
                x_hbm.at[pl.ds((base + i) * SEQ_PER_SUPER, SEQ_PER_SUPER)],
                idx[b])
            for s in range(SEQ_PER_SUPER):
                pltpu.async_copy(
                    tok_hbm.at[idx[b].at[s]], rows[b].at[s], sem_g[b],
                    add=True)

        def stage_b(i, b):
            for s in range(SEQ_PER_SUPER):
                pltpu.make_async_copy(
                    tok_hbm.at[idx[b].at[s]], rows[b].at[s], sem_g[b]).wait()
            pltpu.async_copy(
                rows[b],
                out_hbm.at[pl.ds((base + i) * SEQ_PER_SUPER, SEQ_PER_SUPER)],
                sem_s[b])

        stage_a(0, 0, True)
        stage_a(1, 1, True)
        stage_b(0, 0)

        @pl.loop(0, (supers_per_w - 2) // 2)
        def _(t):
            i = 2 + 2 * t
            stage_a(i, 0, False)
            stage_b(i - 1, 1)
            stage_a(i + 1, 1, False)
            stage_b(i, 0)

        stage_b(supers_per_w - 1, 1)
        wait_store(0)
        wait_store(1)

    return sc_kernel(x32, token_table, pos_table)
